# baseline (speedup 1.0000x reference)
import functools

import jax
import jax.numpy as jnp
from jax import lax
from jax.experimental import pallas as pl
from jax.experimental.pallas import tpu as pltpu
from jax.experimental.pallas import tpu_sc as plsc

_B = 65536
_F = 768

_NUM_CORES = 2
_NUM_SUBCORES = 16
_NW = _NUM_CORES * _NUM_SUBCORES
_RCHUNK = 16


def _sc_body(nchunks, l_hbm, r_hbm, o_hbm,
             l0, l1, r0, r1, o0, o1,
             in_l0, in_l1, in_r0, in_r1, out0, out1):
    span = nchunks * _RCHUNK
    wid = lax.axis_index("s") * _NUM_CORES + lax.axis_index("c")
    base = wid * span

    lbuf = (l0, l1)
    rbuf = (r0, r1)
    obuf = (o0, o1)
    in_l = (in_l0, in_l1)
    in_r = (in_r0, in_r1)
    out = (out0, out1)

    def l_slice(c):
        return l_hbm.at[pl.ds(base + c * _RCHUNK, _RCHUNK), :]

    def r_slice(c):
        return r_hbm.at[pl.ds(base + c * _RCHUNK, _RCHUNK), :]

    def o_slice(c):
        return o_hbm.at[pl.ds(base + c * _RCHUNK, _RCHUNK), :]

    for b in range(2):
        pltpu.async_copy(l_slice(b), lbuf[b], in_l[b])
        pltpu.async_copy(r_slice(b), rbuf[b], in_r[b])

    @pl.loop(0, nchunks, step=2)
    def _ring(g):
        for b in range(2):
            c = g + b
            pltpu.make_async_copy(l_slice(c), lbuf[b], in_l[b]).wait()
            pltpu.make_async_copy(r_slice(c), rbuf[b], in_r[b]).wait()

            @pl.when(g >= 2)
            def _():
                pltpu.make_async_copy(obuf[b], o_slice(c - 2), out[b]).wait()

            for r in range(_RCHUNK):
                @plsc.parallel_loop(0, _F, step=16, unroll=8)
                def _mul(j):
                    obuf[b][r, pl.ds(j, 16)] = (
                        lbuf[b][r, pl.ds(j, 16)] * rbuf[b][r, pl.ds(j, 16)]
                    )

            @pl.when(g < nchunks - 2)
            def _():
                pltpu.async_copy(l_slice(c + 2), lbuf[b], in_l[b])
                pltpu.async_copy(r_slice(c + 2), rbuf[b], in_r[b])

            pltpu.async_copy(obuf[b], o_slice(c), out[b])

    for b in range(2):
        pltpu.make_async_copy(obuf[b], o_slice(nchunks - 2 + b), out[b]).wait()


@functools.cache
def _make_sc_mul(n_rows):
    nchunks = n_rows // (_NW * _RCHUNK)
    assert nchunks * _NW * _RCHUNK == n_rows and nchunks % 2 == 0

    @functools.partial(
        pl.kernel,
        out_type=jax.ShapeDtypeStruct((n_rows, _F), jnp.float32),
        mesh=plsc.VectorSubcoreMesh(core_axis_name="c", subcore_axis_name="s"),
        scratch_types=(
            [pltpu.VMEM((_RCHUNK, _F), jnp.float32)] * 6
            + [pltpu.SemaphoreType.DMA] * 6
        ),
    )
    def sc_mul(l_hbm, r_hbm, o_hbm, *scratch):
        _sc_body(nchunks, l_hbm, r_hbm, o_hbm, *scratch)

    return sc_mul


_TC_BLOCK = 2048
_SC_ROWS = 24576


def _tc_mul_body(l_ref, r_ref, o_ref):
    o_ref[...] = l_ref[...] * r_ref[...]


def _tc_mul_tail(left, right):
    skip = _SC_ROWS // _TC_BLOCK
    n_blocks = _B // _TC_BLOCK - skip
    return pl.pallas_call(
        _tc_mul_body,
        grid=(n_blocks,),
        in_specs=[
            pl.BlockSpec((_TC_BLOCK, _F), lambda i: (i + skip, 0)),
            pl.BlockSpec((_TC_BLOCK, _F), lambda i: (i + skip, 0)),
        ],
        out_specs=pl.BlockSpec((_TC_BLOCK, _F), lambda i: (i + skip, 0)),
        out_shape=jax.ShapeDtypeStruct((_B, _F), left.dtype),
    )(left, right)


def _copy_body(sc_ref, _, o_ref):
    o_ref[...] = sc_ref[...]


def _combine(sc_out, tc_full):
    n_blocks = _SC_ROWS // _TC_BLOCK
    return pl.pallas_call(
        _copy_body,
        grid=(n_blocks,),
        in_specs=[
            pl.BlockSpec((_TC_BLOCK, _F), lambda i: (i, 0)),
            pl.BlockSpec(memory_space=pltpu.MemorySpace.HBM),
        ],
        out_specs=pl.BlockSpec((_TC_BLOCK, _F), lambda i: (i, 0)),
        out_shape=jax.ShapeDtypeStruct((_B, _F), sc_out.dtype),
        input_output_aliases={1: 0},
    )(sc_out, tc_full)


def kernel(left_input, right_input):
    sc_out = _make_sc_mul(_SC_ROWS)(left_input, right_input)
    tc_full = _tc_mul_tail(left_input, right_input)
    return _combine(sc_out, tc_full)


# speedup vs baseline: 1.3866x; 1.3866x over previous
import jax
import jax.numpy as jnp
from jax.experimental import pallas as pl


def _mul_body(l_ref, r_ref, o_ref):
    o_ref[...] = l_ref[...] * r_ref[...]


def kernel(left_input, right_input):
    B, F = left_input.shape
    rows = 1024
    return pl.pallas_call(
        _mul_body,
        grid=(B // rows,),
        in_specs=[
            pl.BlockSpec((rows, F), lambda i: (i, 0)),
            pl.BlockSpec((rows, F), lambda i: (i, 0)),
        ],
        out_specs=pl.BlockSpec((rows, F), lambda i: (i, 0)),
        out_shape=jax.ShapeDtypeStruct((B, F), left_input.dtype),
    )(left_input, right_input)
